# Initial kernel scaffold; baseline (speedup 1.0000x reference)
import functools

import jax
import jax.numpy as jnp
from jax import lax
from jax.experimental import pallas as pl
from jax.experimental.pallas import tpu as pltpu
from jax.experimental.pallas import tpu_sc as plsc

CODEBOOK = 8192
FEAT = 1024
BATCH = 4096

BR = 1024
BC = 1024
NI = BATCH // BR
NJ = CODEBOOK // BC


def _argmin_body(zf_ref, emb_ref, zf2_ref, e2_ref, ind_ref, loss_ref,
                 runv_ref, runi_ref):
    j = pl.program_id(1)
    m = lax.dot_general(
        zf_ref[...], emb_ref[...],
        dimension_numbers=(((1,), (1,)), ((), ())),
        preferred_element_type=jnp.float32,
    )
    dist = (zf2_ref[...] - 2.0 * m) + e2_ref[...]
    bmin = jnp.min(dist, axis=1, keepdims=True)
    lane = lax.broadcasted_iota(jnp.int32, dist.shape, 1)
    bidx = jnp.min(jnp.where(dist == bmin, lane, BC), axis=1, keepdims=True)
    bidx = bidx + j * BC

    @pl.when(j == 0)
    def _():
        runv_ref[...] = bmin
        runi_ref[...] = bidx

    @pl.when(j > 0)
    def _():
        upd = bmin < runv_ref[...]
        runi_ref[...] = jnp.where(upd, bidx, runi_ref[...])
        runv_ref[...] = jnp.where(upd, bmin, runv_ref[...])

    @pl.when(j == NJ - 1)
    def _():
        ind_ref[...] = runi_ref[...]
        s = jnp.sum(runv_ref[...])
        i = pl.program_id(0)

        @pl.when(i == 0)
        def _():
            loss_ref[0, 0] = s

        @pl.when(i > 0)
        def _():
            loss_ref[0, 0] = loss_ref[0, 0] + s


def _argmin_dist(zf, embedding, zf2, e2):
    return pl.pallas_call(
        _argmin_body,
        grid=(NI, NJ),
        in_specs=[
            pl.BlockSpec((BR, FEAT), lambda i, j: (i, 0)),
            pl.BlockSpec((BC, FEAT), lambda i, j: (j, 0)),
            pl.BlockSpec((BR, 1), lambda i, j: (i, 0)),
            pl.BlockSpec((1, BC), lambda i, j: (0, j)),
        ],
        out_specs=[
            pl.BlockSpec((BR, 1), lambda i, j: (i, 0)),
            pl.BlockSpec((1, 1), lambda i, j: (0, 0)),
        ],
        out_shape=[
            jax.ShapeDtypeStruct((BATCH, 1), jnp.int32),
            jax.ShapeDtypeStruct((1, 1), jnp.float32),
        ],
        scratch_shapes=[
            pltpu.VMEM((BR, 1), jnp.float32),
            pltpu.VMEM((BR, 1), jnp.int32),
        ],
        compiler_params=pltpu.CompilerParams(
            dimension_semantics=("arbitrary", "arbitrary"),
        ),
    )(zf, embedding, zf2, e2)


NW = 32
BPW = BATCH // NW
CH = 64
NCH = BPW // CH


def _gather_body(emb_hbm, idx_hbm, out_hbm, idx_v, rows_v, sem):
    wid = lax.axis_index("s") * 2 + lax.axis_index("c")
    base = wid * BPW
    pltpu.sync_copy(idx_hbm.at[pl.ds(base, BPW)], idx_v)
    for c in range(NCH):
        pltpu.async_copy(
            emb_hbm.at[idx_v.at[pl.ds(c * CH, CH)]], rows_v, sem
        ).wait()
        pltpu.sync_copy(rows_v, out_hbm.at[pl.ds(base + c * CH, CH)])


def _sc_gather(embedding, ind):
    mesh = plsc.VectorSubcoreMesh(core_axis_name="c", subcore_axis_name="s")
    return pl.kernel(
        _gather_body,
        mesh=mesh,
        out_type=jax.ShapeDtypeStruct((BATCH, FEAT), jnp.float32),
        scratch_types=[
            pltpu.VMEM((BPW,), jnp.int32),
            pltpu.VMEM((CH, FEAT), jnp.float32),
            pltpu.SemaphoreType.DMA,
        ],
    )(embedding, ind)


def kernel(z, embedding):
    Bb, N, D = z.shape
    zf = z.reshape(Bb, N * D)
    zf2 = jnp.sum(zf ** 2, axis=1, keepdims=True)
    e2 = jnp.sum(embedding ** 2, axis=1, keepdims=True).T
    ind2d, losssum = _argmin_dist(zf, embedding, zf2, e2)
    ind = ind2d.reshape(Bb)
    z_q = _sc_gather(embedding, ind)
    latent_loss = losssum[0, 0] * (12.5 / (Bb * N * D))
    return (z_q, latent_loss)


# speedup vs baseline: 1.2747x; 1.2747x over previous
import functools

import jax
import jax.numpy as jnp
from jax import lax
from jax.experimental import pallas as pl
from jax.experimental.pallas import tpu as pltpu
from jax.experimental.pallas import tpu_sc as plsc

CODEBOOK = 8192
FEAT = 1024
BATCH = 4096

BR = 1024
BC = 1024
NI = BATCH // BR
NJ = CODEBOOK // BC


def _argmin_body(zf_ref, emb_ref, zf2_ref, e2_ref, ind_ref, loss_ref,
                 runv_ref, runi_ref):
    j = pl.program_id(1)
    m = lax.dot_general(
        zf_ref[...], emb_ref[...],
        dimension_numbers=(((1,), (1,)), ((), ())),
        preferred_element_type=jnp.float32,
    )
    dist = (zf2_ref[...] - 2.0 * m) + e2_ref[...]
    bmin = jnp.min(dist, axis=1, keepdims=True)
    lane = lax.broadcasted_iota(jnp.int32, dist.shape, 1)
    bidx = jnp.min(jnp.where(dist == bmin, lane, BC), axis=1, keepdims=True)
    bidx = bidx + j * BC

    @pl.when(j == 0)
    def _():
        runv_ref[...] = bmin
        runi_ref[...] = bidx

    @pl.when(j > 0)
    def _():
        upd = bmin < runv_ref[...]
        runi_ref[...] = jnp.where(upd, bidx, runi_ref[...])
        runv_ref[...] = jnp.where(upd, bmin, runv_ref[...])

    @pl.when(j == NJ - 1)
    def _():
        ind_ref[...] = runi_ref[...]
        s = jnp.reshape(jnp.sum(runv_ref[...]), (1, 1))
        i = pl.program_id(0)

        @pl.when(i == 0)
        def _():
            loss_ref[...] = s

        @pl.when(i > 0)
        def _():
            loss_ref[...] = loss_ref[...] + s


def _argmin_dist(zf, embedding, zf2, e2):
    return pl.pallas_call(
        _argmin_body,
        grid=(NI, NJ),
        in_specs=[
            pl.BlockSpec((BR, FEAT), lambda i, j: (i, 0)),
            pl.BlockSpec((BC, FEAT), lambda i, j: (j, 0)),
            pl.BlockSpec((BR, 1), lambda i, j: (i, 0)),
            pl.BlockSpec((1, BC), lambda i, j: (0, j)),
        ],
        out_specs=[
            pl.BlockSpec((BR, 1), lambda i, j: (i, 0)),
            pl.BlockSpec((1, 1), lambda i, j: (0, 0)),
        ],
        out_shape=[
            jax.ShapeDtypeStruct((BATCH, 1), jnp.int32),
            jax.ShapeDtypeStruct((1, 1), jnp.float32),
        ],
        scratch_shapes=[
            pltpu.VMEM((BR, 1), jnp.float32),
            pltpu.VMEM((BR, 1), jnp.int32),
        ],
        compiler_params=pltpu.CompilerParams(
            dimension_semantics=("arbitrary", "arbitrary"),
        ),
    )(zf, embedding, zf2, e2)


NW = 32
BPW = BATCH // NW
CH = 64
NCH = BPW // CH


def _gather_body(emb_hbm, idx_hbm, out_hbm, idx_v, rows_v, sem):
    wid = lax.axis_index("s") * 2 + lax.axis_index("c")
    base = wid * BPW
    pltpu.sync_copy(idx_hbm.at[pl.ds(base, BPW)], idx_v)
    for c in range(NCH):
        pltpu.async_copy(
            emb_hbm.at[idx_v.at[pl.ds(c * CH, CH)]], rows_v, sem
        ).wait()
        pltpu.sync_copy(rows_v, out_hbm.at[pl.ds(base + c * CH, CH)])


def _sc_gather(embedding, ind):
    mesh = plsc.VectorSubcoreMesh(core_axis_name="c", subcore_axis_name="s")
    return pl.kernel(
        _gather_body,
        mesh=mesh,
        out_type=jax.ShapeDtypeStruct((BATCH, FEAT), jnp.float32),
        scratch_types=[
            pltpu.VMEM((BPW,), jnp.int32),
            pltpu.VMEM((CH, FEAT), jnp.float32),
            pltpu.SemaphoreType.DMA,
        ],
    )(embedding, ind)


def kernel(z, embedding):
    Bb, N, D = z.shape
    zf = z.reshape(Bb, N * D)
    zf2 = jnp.sum(zf ** 2, axis=1, keepdims=True)
    e2 = jnp.sum(embedding ** 2, axis=1, keepdims=True).T
    ind2d, losssum = _argmin_dist(zf, embedding, zf2, e2)
    ind = ind2d.reshape(Bb)
    z_q = _sc_gather(embedding, ind)
    latent_loss = losssum[0, 0] * (12.5 / (Bb * N * D))
    return (z_q, latent_loss)
